# Initial kernel scaffold; baseline (speedup 1.0000x reference)
import jax
import jax.numpy as jnp
from jax.experimental import pallas as pl


def kernel(x, role_labels, emb):
    raise NotImplementedError("write your pallas kernel here")


# speedup vs baseline: 2.5941x; 2.5941x over previous
import math

import jax
import jax.numpy as jnp
from jax.experimental import pallas as pl

D_MODEL_K = 1024
ROWS_PER_BLOCK = 512
INV_SQRT_D = 1.0 / math.sqrt(D_MODEL_K)


def _body(lab_ref, x_ref, emb_ref, o_ref):
    lab = lab_ref[0]
    r = lab.shape[-1]
    ohT = (jax.lax.broadcasted_iota(jnp.int32, (3, r), 0) == lab).astype(jnp.float32)
    rows = jax.lax.dot_general(
        ohT, emb_ref[...],
        dimension_numbers=(((0,), (0,)), ((), ())),
        preferred_element_type=jnp.float32,
    )
    o_ref[...] = x_ref[...] + rows * INV_SQRT_D


def kernel(x, role_labels, emb):
    b, s, d = x.shape
    n_rows = b * s
    g = n_rows // ROWS_PER_BLOCK
    x2 = x.reshape(n_rows, d)
    lab3 = role_labels.astype(jnp.int32).reshape(g, 1, ROWS_PER_BLOCK)
    out = pl.pallas_call(
        _body,
        grid=(g,),
        in_specs=[
            pl.BlockSpec((1, 1, ROWS_PER_BLOCK), lambda i: (i, 0, 0)),
            pl.BlockSpec((ROWS_PER_BLOCK, d), lambda i: (i, 0)),
            pl.BlockSpec((3, d), lambda i: (0, 0)),
        ],
        out_specs=pl.BlockSpec((ROWS_PER_BLOCK, d), lambda i: (i, 0)),
        out_shape=jax.ShapeDtypeStruct((n_rows, d), jnp.float32),
    )(lab3, x2, emb)
    return out.reshape(b, s, d)


# speedup vs baseline: 3.0119x; 1.1610x over previous
import math

import jax
import jax.numpy as jnp
from jax.experimental import pallas as pl

D_MODEL_K = 1024
ROWS_PER_BLOCK = 1024
INV_SQRT_D = 1.0 / math.sqrt(D_MODEL_K)


def _body(lab_ref, x_ref, emb_ref, o_ref):
    lab = lab_ref[0]
    r = lab.shape[-1]
    ohT = (jax.lax.broadcasted_iota(jnp.int32, (3, r), 0) == lab).astype(jnp.float32)
    rows = jax.lax.dot_general(
        ohT, emb_ref[...],
        dimension_numbers=(((0,), (0,)), ((), ())),
        preferred_element_type=jnp.float32,
    )
    o_ref[...] = x_ref[...] + rows * INV_SQRT_D


def kernel(x, role_labels, emb):
    b, s, d = x.shape
    n_rows = b * s
    g = n_rows // ROWS_PER_BLOCK
    x2 = x.reshape(n_rows, d)
    lab3 = role_labels.astype(jnp.int32).reshape(g, 1, ROWS_PER_BLOCK)
    out = pl.pallas_call(
        _body,
        grid=(g,),
        in_specs=[
            pl.BlockSpec((1, 1, ROWS_PER_BLOCK), lambda i: (i, 0, 0)),
            pl.BlockSpec((ROWS_PER_BLOCK, d), lambda i: (i, 0)),
            pl.BlockSpec((3, d), lambda i: (0, 0)),
        ],
        out_specs=pl.BlockSpec((ROWS_PER_BLOCK, d), lambda i: (i, 0)),
        out_shape=jax.ShapeDtypeStruct((n_rows, d), jnp.float32),
    )(lab3, x2, emb)
    return out.reshape(b, s, d)


# speedup vs baseline: 3.0893x; 1.0257x over previous
import math

import jax
import jax.numpy as jnp
from jax.experimental import pallas as pl

D_MODEL_K = 1024
ROWS_PER_BLOCK = 2048
INV_SQRT_D = 1.0 / math.sqrt(D_MODEL_K)


def _body(lab_ref, x_ref, emb_ref, o_ref):
    lab = lab_ref[0]
    r = lab.shape[-1]
    ohT = (jax.lax.broadcasted_iota(jnp.int32, (3, r), 0) == lab).astype(jnp.float32)
    rows = jax.lax.dot_general(
        ohT, emb_ref[...],
        dimension_numbers=(((0,), (0,)), ((), ())),
        preferred_element_type=jnp.float32,
    )
    o_ref[...] = x_ref[...] + rows * INV_SQRT_D


def kernel(x, role_labels, emb):
    b, s, d = x.shape
    n_rows = b * s
    g = n_rows // ROWS_PER_BLOCK
    x2 = x.reshape(n_rows, d)
    lab3 = role_labels.astype(jnp.int32).reshape(g, 1, ROWS_PER_BLOCK)
    out = pl.pallas_call(
        _body,
        grid=(g,),
        in_specs=[
            pl.BlockSpec((1, 1, ROWS_PER_BLOCK), lambda i: (i, 0, 0)),
            pl.BlockSpec((ROWS_PER_BLOCK, d), lambda i: (i, 0)),
            pl.BlockSpec((3, d), lambda i: (0, 0)),
        ],
        out_specs=pl.BlockSpec((ROWS_PER_BLOCK, d), lambda i: (i, 0)),
        out_shape=jax.ShapeDtypeStruct((n_rows, d), jnp.float32),
    )(lab3, x2, emb)
    return out.reshape(b, s, d)
